# Initial kernel scaffold; baseline (speedup 1.0000x reference)
import jax
import jax.numpy as jnp
from jax.experimental import pallas as pl


def kernel(x, wih0, whh0, b0, wih1, whh1, b1, wmid, bmid, wfc, bfc):
    raise NotImplementedError("write your pallas kernel here")


# speedup vs baseline: 1.0414x; 1.0414x over previous
import functools

import jax
import jax.numpy as jnp
from jax.experimental import pallas as pl
from jax.experimental.pallas import tpu as pltpu


def _proj_kernel(xt_ref, wih0_ref, b0_ref, out_ref):
    Tc, B, I = xt_ref.shape
    rows = xt_ref[...].reshape(Tc * B, I)
    g = jnp.dot(rows, wih0_ref[...], preferred_element_type=jnp.float32)
    g = g + b0_ref[...]
    out_ref[...] = g.reshape(Tc, B, g.shape[-1])


def _gate_act(gates, c, G):
    sfo = jax.nn.sigmoid(gates[:, :3 * G])
    g_g = jnp.tanh(gates[:, 3 * G:])
    i_g = sfo[:, 0 * G:1 * G]
    f_g = sfo[:, 1 * G:2 * G]
    o_g = sfo[:, 2 * G:3 * G]
    c_new = f_g * c + i_g * g_g
    h_new = o_g * jnp.tanh(c_new)
    return h_new, c_new


def _rnn_chunk_kernel(nc, xg_ref, whh0_ref, w1_ref, b1_ref, wout_ref,
                      bout_ref, out_ref, h0_ref, c0_ref, h1_ref, c1_ref):
    C, B, G4 = xg_ref.shape
    G = G4 // 4
    i = pl.program_id(0)

    @pl.when(i == 0)
    def _init():
        z = jnp.zeros((B, G), jnp.float32)
        h0_ref[...] = z
        c0_ref[...] = z
        h1_ref[...] = z
        c1_ref[...] = z

    whh0 = whh0_ref[...]
    w1 = w1_ref[...]
    b1 = jnp.broadcast_to(b1_ref[...], (B, G4))

    h0 = h0_ref[...]
    c0 = c0_ref[...]
    h1 = h1_ref[...]
    c1 = c1_ref[...]
    for t in range(C):
        g0 = xg_ref[t] + jnp.dot(h0, whh0, preferred_element_type=jnp.float32)
        h0, c0 = _gate_act(g0, c0, G)
        g1 = jnp.dot(jnp.concatenate([h0, h1], axis=-1), w1,
                     preferred_element_type=jnp.float32) + b1
        h1, c1 = _gate_act(g1, c1, G)
    h0_ref[...] = h0
    c0_ref[...] = c0
    h1_ref[...] = h1
    c1_ref[...] = c1

    @pl.when(i == nc - 1)
    def _head():
        r = jnp.maximum(h1, 0.0)
        out_ref[...] = (jnp.dot(r, wout_ref[...],
                                preferred_element_type=jnp.float32)
                        + bout_ref[...])


@functools.partial(jax.jit, static_argnames=())
def kernel(x, wih0, whh0, b0, wih1, whh1, b1, wmid, bmid, wfc, bfc):
    B, T, I = x.shape
    G = whh0.shape[0]
    G4 = 4 * G
    O = wfc.shape[1]

    def reorder(w):
        return jnp.concatenate(
            [w[..., 0:G], w[..., G:2 * G], w[..., 3 * G:4 * G],
             w[..., 2 * G:3 * G]], axis=-1)

    wih0_r = reorder(wih0)
    whh0_r = reorder(whh0)
    b0_r = reorder(b0)
    w1 = jnp.concatenate([reorder(wih1), reorder(whh1)], axis=0)
    b1_r = reorder(b1)
    wout = wmid @ wfc
    bout = bmid @ wfc + bfc

    xt = jnp.transpose(x, (1, 0, 2))
    PC = 4
    Tp = T // PC
    xg0 = pl.pallas_call(
        _proj_kernel,
        out_shape=jax.ShapeDtypeStruct((T, B, G4), jnp.float32),
        grid=(PC,),
        in_specs=[
            pl.BlockSpec((Tp, B, I), lambda i: (i, 0, 0)),
            pl.BlockSpec((I, G4), lambda i: (0, 0)),
            pl.BlockSpec((1, G4), lambda i: (0, 0)),
        ],
        out_specs=pl.BlockSpec((Tp, B, G4), lambda i: (i, 0, 0)),
        compiler_params=pltpu.CompilerParams(
            dimension_semantics=("parallel",)),
    )(xt, wih0_r, b0_r)

    NC = 4
    C = T // NC
    out = pl.pallas_call(
        functools.partial(_rnn_chunk_kernel, NC),
        out_shape=jax.ShapeDtypeStruct((B, O), jnp.float32),
        grid=(NC,),
        in_specs=[
            pl.BlockSpec((C, B, G4), lambda i: (i, 0, 0)),
            pl.BlockSpec((G, G4), lambda i: (0, 0)),
            pl.BlockSpec((2 * G, G4), lambda i: (0, 0)),
            pl.BlockSpec((1, G4), lambda i: (0, 0)),
            pl.BlockSpec((G, O), lambda i: (0, 0)),
            pl.BlockSpec((1, O), lambda i: (0, 0)),
        ],
        out_specs=pl.BlockSpec((B, O), lambda i: (0, 0)),
        scratch_shapes=[
            pltpu.VMEM((B, G), jnp.float32),
            pltpu.VMEM((B, G), jnp.float32),
            pltpu.VMEM((B, G), jnp.float32),
            pltpu.VMEM((B, G), jnp.float32),
        ],
        compiler_params=pltpu.CompilerParams(
            dimension_semantics=("arbitrary",)),
    )(xg0, whh0_r, w1, b1_r, wout, bout)
    return out


# speedup vs baseline: 1.5645x; 1.5022x over previous
import functools

import jax
import jax.numpy as jnp
from jax.experimental import pallas as pl
from jax.experimental.pallas import tpu as pltpu


def _proj_kernel(x_ref, wih0_ref, b0_ref, out_ref):
    Bc, T, I = x_ref.shape
    rows = x_ref[...].reshape(Bc * T, I)
    g = jnp.dot(rows, wih0_ref[...], preferred_element_type=jnp.float32)
    g = g + b0_ref[...]
    out_ref[...] = g.reshape(Bc, T, g.shape[-1])


def _gate_act(gates, c, G):
    sif = jax.nn.sigmoid(gates[:, :2 * G])
    i_g = sif[:, :G]
    f_g = sif[:, G:]
    g_g = jnp.tanh(gates[:, 2 * G:3 * G])
    o_g = jax.nn.sigmoid(gates[:, 3 * G:])
    c_new = f_g * c + i_g * g_g
    h_new = o_g * jnp.tanh(c_new)
    return h_new, c_new


def _rnn_chunk_kernel(nc, xg_ref, whh0_ref, wih1_ref, whh1_ref, b1_ref,
                      wmid_ref, bmid_ref, wfc_ref, bfc_ref, out_ref,
                      h0_ref, c0_ref, h1_ref, c1_ref):
    B, C, G4 = xg_ref.shape
    G = G4 // 4
    i = pl.program_id(0)

    @pl.when(i == 0)
    def _init():
        z = jnp.zeros((B, G), jnp.float32)
        h0_ref[...] = z
        c0_ref[...] = z
        h1_ref[...] = z
        c1_ref[...] = z

    whh0 = whh0_ref[...]
    wih1 = wih1_ref[...]
    whh1 = whh1_ref[...]
    b1 = jnp.broadcast_to(b1_ref[...], (B, G4))

    h0 = h0_ref[...]
    c0 = c0_ref[...]
    h1 = h1_ref[...]
    c1 = c1_ref[...]
    for t in range(C):
        g0 = xg_ref[:, t, :] + jnp.dot(h0, whh0,
                                       preferred_element_type=jnp.float32)
        h0, c0 = _gate_act(g0, c0, G)
        g1 = (jnp.dot(h0, wih1, preferred_element_type=jnp.float32)
              + jnp.dot(h1, whh1, preferred_element_type=jnp.float32) + b1)
        h1, c1 = _gate_act(g1, c1, G)
    h0_ref[...] = h0
    c0_ref[...] = c0
    h1_ref[...] = h1
    c1_ref[...] = c1

    @pl.when(i == nc - 1)
    def _head():
        r = jnp.maximum(h1, 0.0)
        mid = (jnp.dot(r, wmid_ref[...], preferred_element_type=jnp.float32)
               + bmid_ref[...])
        out_ref[...] = (jnp.dot(mid, wfc_ref[...],
                                preferred_element_type=jnp.float32)
                        + bfc_ref[...])


@jax.jit
def kernel(x, wih0, whh0, b0, wih1, whh1, b1, wmid, bmid, wfc, bfc):
    B, T, I = x.shape
    G = whh0.shape[0]
    G4 = 4 * G
    H = wmid.shape[1]
    O = wfc.shape[1]

    PC = 4
    Bc = B // PC
    xg0 = pl.pallas_call(
        _proj_kernel,
        out_shape=jax.ShapeDtypeStruct((B, T, G4), jnp.float32),
        grid=(PC,),
        in_specs=[
            pl.BlockSpec((Bc, T, I), lambda i: (i, 0, 0)),
            pl.BlockSpec((I, G4), lambda i: (0, 0)),
            pl.BlockSpec((1, G4), lambda i: (0, 0)),
        ],
        out_specs=pl.BlockSpec((Bc, T, G4), lambda i: (i, 0, 0)),
        compiler_params=pltpu.CompilerParams(
            dimension_semantics=("parallel",)),
    )(x, wih0, b0)

    NC = 4
    C = T // NC
    out = pl.pallas_call(
        functools.partial(_rnn_chunk_kernel, NC),
        out_shape=jax.ShapeDtypeStruct((B, O), jnp.float32),
        grid=(NC,),
        in_specs=[
            pl.BlockSpec((B, C, G4), lambda i: (0, i, 0)),
            pl.BlockSpec((G, G4), lambda i: (0, 0)),
            pl.BlockSpec((G, G4), lambda i: (0, 0)),
            pl.BlockSpec((G, G4), lambda i: (0, 0)),
            pl.BlockSpec((1, G4), lambda i: (0, 0)),
            pl.BlockSpec((G, H), lambda i: (0, 0)),
            pl.BlockSpec((1, H), lambda i: (0, 0)),
            pl.BlockSpec((H, O), lambda i: (0, 0)),
            pl.BlockSpec((1, O), lambda i: (0, 0)),
        ],
        out_specs=pl.BlockSpec((B, O), lambda i: (0, 0)),
        scratch_shapes=[
            pltpu.VMEM((B, G), jnp.float32),
            pltpu.VMEM((B, G), jnp.float32),
            pltpu.VMEM((B, G), jnp.float32),
            pltpu.VMEM((B, G), jnp.float32),
        ],
        compiler_params=pltpu.CompilerParams(
            dimension_semantics=("arbitrary",)),
    )(xg0, whh0, wih1, whh1, b1, wmid, bmid, wfc, bfc)
    return out


# speedup vs baseline: 1.7125x; 1.0946x over previous
import jax
import jax.numpy as jnp
from jax.experimental import pallas as pl
from jax.experimental.pallas import tpu as pltpu


def _gate_act(gates, c, G):
    sif = jax.nn.sigmoid(gates[:, :2 * G])
    i_g = sif[:, :G]
    f_g = sif[:, G:]
    g_g = jnp.tanh(gates[:, 2 * G:3 * G])
    o_g = jax.nn.sigmoid(gates[:, 3 * G:])
    c_new = f_g * c + i_g * g_g
    h_new = o_g * jnp.tanh(c_new)
    return h_new, c_new


def _fused_kernel(x_ref, wih0_ref, b0_ref, whh0_ref, wih1_ref, whh1_ref,
                  b1_ref, wmid_ref, bmid_ref, wfc_ref, bfc_ref, out_ref,
                  xg_ref):
    B, T, I = x_ref.shape
    Q, _, TL, G4 = xg_ref.shape
    G = G4 // 4

    wih0 = wih0_ref[...]
    b0 = b0_ref[...]
    for q in range(Q):
        rows = x_ref[:, q * TL:(q + 1) * TL, :].reshape(B * TL, I)
        g = jnp.dot(rows, wih0, preferred_element_type=jnp.float32) + b0
        xg_ref[q] = g.reshape(B, TL, G4)

    whh0 = whh0_ref[...]
    wih1 = wih1_ref[...]
    whh1 = whh1_ref[...]
    b1 = jnp.broadcast_to(b1_ref[...], (B, G4))

    z = jnp.zeros((B, G), jnp.float32)
    h0, c0, h1, c1 = z, z, z, z
    for t in range(T):
        g0 = xg_ref[t // TL, :, t % TL, :] + jnp.dot(
            h0, whh0, preferred_element_type=jnp.float32)
        h0, c0 = _gate_act(g0, c0, G)
        g1 = (jnp.dot(h0, wih1, preferred_element_type=jnp.float32)
              + jnp.dot(h1, whh1, preferred_element_type=jnp.float32) + b1)
        h1, c1 = _gate_act(g1, c1, G)

    r = jnp.maximum(h1, 0.0)
    mid = (jnp.dot(r, wmid_ref[...], preferred_element_type=jnp.float32)
           + bmid_ref[...])
    out_ref[...] = (jnp.dot(mid, wfc_ref[...],
                            preferred_element_type=jnp.float32)
                    + bfc_ref[...])


@jax.jit
def kernel(x, wih0, whh0, b0, wih1, whh1, b1, wmid, bmid, wfc, bfc):
    B, T, I = x.shape
    G = whh0.shape[0]
    G4 = 4 * G
    H = wmid.shape[1]
    O = wfc.shape[1]
    Q, TL = 4, T // 4

    const = lambda i: (0, 0)
    out = pl.pallas_call(
        _fused_kernel,
        out_shape=jax.ShapeDtypeStruct((B, O), jnp.float32),
        grid=(1,),
        in_specs=[
            pl.BlockSpec((B, T, I), lambda i: (0, 0, 0)),
            pl.BlockSpec((I, G4), const),
            pl.BlockSpec((1, G4), const),
            pl.BlockSpec((G, G4), const),
            pl.BlockSpec((G, G4), const),
            pl.BlockSpec((G, G4), const),
            pl.BlockSpec((1, G4), const),
            pl.BlockSpec((G, H), const),
            pl.BlockSpec((1, H), const),
            pl.BlockSpec((H, O), const),
            pl.BlockSpec((1, O), const),
        ],
        out_specs=pl.BlockSpec((B, O), const),
        scratch_shapes=[
            pltpu.VMEM((Q, B, TL, G4), jnp.float32),
        ],
        compiler_params=pltpu.CompilerParams(
            dimension_semantics=("arbitrary",)),
    )(x, wih0, b0, whh0, wih1, whh1, b1, wmid, bmid, wfc, bfc)
    return out
